# baseline (speedup 1.0000x reference)
import functools

import jax
import jax.numpy as jnp
from jax import lax
from jax.experimental import pallas as pl
from jax.experimental.pallas import tpu as pltpu
from jax.experimental.pallas import tpu_sc as plsc

_N = 10000
_E = 320000
_D = 128
_H = 128
_L = 3

_NC = 2
_NS = 16
_NW = _NC * _NS
_EPW = _E // _NW
_CHUNK = 80
_STEPS = 128
_EPWP = _STEPS * _CHUNK
_NP = 10240
_RPS = _NP // _NS
_NIDX = 2


def _sc_partials_body(h_hbm, src_hbm, dst_hbm, out_hbm,
                      sidx, didx, rows, acc, gsem, isS, isD):
    cid = lax.axis_index("c")
    sid = lax.axis_index("s")
    wid = sid * _NC + cid
    row0 = sid * _RPS
    ebase = wid * _EPWP

    def idx_start(c, k):
        off = ebase + c * _CHUNK
        pltpu.async_copy(src_hbm.at[pl.ds(off, _CHUNK)], sidx.at[k, 0],
                         isS.at[k])
        pltpu.async_copy(dst_hbm.at[pl.ds(off, _CHUNK)], didx.at[k, 0],
                         isD.at[k])

    def sidx_wait(c, k):
        off = ebase + c * _CHUNK
        pltpu.make_async_copy(src_hbm.at[pl.ds(off, _CHUNK)], sidx.at[k, 0],
                              isS.at[k]).wait()

    def didx_wait(c, k):
        off = ebase + c * _CHUNK
        pltpu.make_async_copy(dst_hbm.at[pl.ds(off, _CHUNK)], didx.at[k, 0],
                              isD.at[k]).wait()

    @pl.loop(0, _CHUNK)
    def _zr(r):
        @pl.loop(0, _H, step=16)
        def _zc(c):
            rows[r, pl.ds(c, 16)] = jnp.zeros((16,), jnp.float32)

    @pl.loop(0, _RPS, step=_CHUNK)
    def _za(r):
        pltpu.sync_copy(rows, acc.at[pl.ds(row0 + r, _CHUNK)])

    plsc.subcore_barrier()

    for k in range(_NIDX):
        idx_start(k, k)

    @pl.loop(0, _STEPS - _NIDX, step=_NIDX)
    def _main(s):
        for k in range(_NIDX):
            c = s + k
            sidx_wait(c, k)
            didx_wait(c, k)
            pltpu.async_copy(h_hbm.at[sidx.at[k, 0]], rows, gsem).wait()
            pltpu.sync_copy(rows, acc.at[didx.at[k, 0]], add=True)
            idx_start(c + _NIDX, k)

    for k in range(_NIDX):
        c = _STEPS - _NIDX + k
        sidx_wait(c, k)
        didx_wait(c, k)
        pltpu.async_copy(h_hbm.at[sidx.at[k, 0]], rows, gsem).wait()
        pltpu.sync_copy(rows, acc.at[didx.at[k, 0]], add=True)

    plsc.subcore_barrier()

    pltpu.sync_copy(acc.at[pl.ds(row0, _RPS)],
                    out_hbm.at[cid, pl.ds(row0, _RPS)])


@jax.jit
def _sc_partials(h, src1, dst1):
    kern = pl.kernel(
        _sc_partials_body,
        out_type=jax.ShapeDtypeStruct((_NC, _NP, _H), jnp.float32),
        mesh=plsc.VectorSubcoreMesh(core_axis_name="c", subcore_axis_name="s"),
        scratch_types=[
            pltpu.VMEM((_NIDX, 1, _CHUNK), jnp.int32),
            pltpu.VMEM((_NIDX, 1, _CHUNK), jnp.int32),
            pltpu.VMEM((_CHUNK, _H), jnp.float32),
            pltpu.VMEM_SHARED((_NP, _H), jnp.float32),
            pltpu.SemaphoreType.DMA,
            pltpu.SemaphoreType.DMA((_NIDX,)),
            pltpu.SemaphoreType.DMA((_NIDX,)),
        ],
    )
    return kern(h, src1, dst1)


def _enc_body(x_ref, w_ref, b_ref, o_ref):
    o_ref[...] = (
        jnp.dot(x_ref[...], w_ref[...], preferred_element_type=jnp.float32)
        + b_ref[...]
    )


@jax.jit
def _encode(x, Wenc, benc):
    return pl.pallas_call(
        _enc_body,
        out_shape=jax.ShapeDtypeStruct((_N, _H), jnp.float32),
    )(x, Wenc, benc.reshape(1, _H))


def _bn_relu(z, g, b):
    m = jnp.mean(z, axis=0, keepdims=True)
    v = jnp.mean(jnp.square(z - m), axis=0, keepdims=True)
    z = g * (z - m) / jnp.sqrt(v + 1e-5) + b
    return jnp.maximum(z, 0.0)


def _gin_mlp(h_ref, p_ref, w1_ref, b1_ref, gm_ref, bm_ref,
             w2_ref, b2_ref, go_ref, bo_ref, sc_ref):
    h = h_ref[...]
    z = sc_ref[...] * h + (p_ref[0, : _N] + p_ref[1, : _N])
    z = jnp.dot(z, w1_ref[...], preferred_element_type=jnp.float32) + b1_ref[...]
    z = _bn_relu(z, gm_ref[...], bm_ref[...])
    z = jnp.dot(z, w2_ref[...], preferred_element_type=jnp.float32) + b2_ref[...]
    z = _bn_relu(z, go_ref[...], bo_ref[...])
    return z + h


def _layer_body(h_ref, p_ref, w1_ref, b1_ref, gm_ref, bm_ref,
                w2_ref, b2_ref, go_ref, bo_ref, sc_ref, o_ref):
    o_ref[...] = _gin_mlp(h_ref, p_ref, w1_ref, b1_ref, gm_ref, bm_ref,
                          w2_ref, b2_ref, go_ref, bo_ref, sc_ref)


def _last_body(h_ref, p_ref, w1_ref, b1_ref, gm_ref, bm_ref,
               w2_ref, b2_ref, go_ref, bo_ref, sc_ref,
               wo_ref, bo2_ref, o_ref):
    hn = _gin_mlp(h_ref, p_ref, w1_ref, b1_ref, gm_ref, bm_ref,
                  w2_ref, b2_ref, go_ref, bo_ref, sc_ref)
    o_ref[...] = (
        jnp.dot(hn, wo_ref[...], preferred_element_type=jnp.float32)
        + bo2_ref[...]
    )


@jax.jit
def _layer(*args):
    return pl.pallas_call(
        _layer_body,
        out_shape=jax.ShapeDtypeStruct((_N, _H), jnp.float32),
    )(*args)


@jax.jit
def _last(*args):
    return pl.pallas_call(
        _last_body,
        out_shape=jax.ShapeDtypeStruct((_N, 1), jnp.float32),
    )(*args)


def kernel(x, edge_index, Wenc, benc, W1, b1, g_mid, bt_mid, W2, b2,
           eps, g_out, bt_out, Wout, bout):
    pad = _EPWP - _EPW
    src1 = jnp.concatenate(
        [edge_index[0].astype(jnp.int32).reshape(_NW, _EPW),
         jnp.zeros((_NW, pad), jnp.int32)], axis=1).reshape(_NW * _EPWP)
    dst1 = jnp.concatenate(
        [edge_index[1].astype(jnp.int32).reshape(_NW, _EPW),
         jnp.full((_NW, pad), _N, jnp.int32)], axis=1).reshape(_NW * _EPWP)
    h = _encode(x, Wenc, benc)
    for i in range(_L):
        parts = _sc_partials(h, src1, dst1)
        sc = (1.0 + eps[i]) * jnp.ones((1, _H), jnp.float32)
        args = (h, parts, W1[i], b1[i].reshape(1, -1),
                g_mid[i].reshape(1, -1), bt_mid[i].reshape(1, -1),
                W2[i], b2[i].reshape(1, -1),
                g_out[i].reshape(1, -1), bt_out[i].reshape(1, -1), sc)
        if i < _L - 1:
            h = _layer(*args)
        else:
            out = _last(*args, Wout, bout.reshape(1, 1))
    return out


# speedup vs baseline: 1.5982x; 1.5982x over previous
import functools

import jax
import jax.numpy as jnp
from jax import lax
from jax.experimental import pallas as pl
from jax.experimental.pallas import tpu as pltpu
from jax.experimental.pallas import tpu_sc as plsc

_N = 10000
_E = 320000
_D = 128
_H = 128
_L = 3

_NC = 2
_NS = 16
_NW = _NC * _NS
_EPW = _E // _NW
_CHUNK = 80
_STEPS = 126
_EPWP = _STEPS * _CHUNK
_NP = 10240
_RPS = _NP // _NS
_ZROWS = 128


def _sc_partials_body(h_hbm, src_hbm, dst_hbm, out_hbm,
                      sA, dA, sB, dB, rows, zv, acc,
                      gsem, semSA, semDA, semSB, semDB):
    cid = lax.axis_index("c")
    sid = lax.axis_index("s")
    wid = sid * _NC + cid
    row0 = sid * _RPS
    ebase = wid * _EPWP

    def idx_start(off, sidx, didx, semS, semD):
        pltpu.async_copy(src_hbm.at[pl.ds(off, _CHUNK)], sidx, semS)
        pltpu.async_copy(dst_hbm.at[pl.ds(off, _CHUNK)], didx, semD)

    def idx_wait(off, sidx, didx, semS, semD):
        pltpu.make_async_copy(src_hbm.at[pl.ds(off, _CHUNK)], sidx,
                              semS).wait()
        pltpu.make_async_copy(dst_hbm.at[pl.ds(off, _CHUNK)], didx,
                              semD).wait()

    def process(off, sidx, didx, semS, semD):
        idx_wait(off, sidx, didx, semS, semD)
        pltpu.async_copy(h_hbm.at[sidx], rows, gsem).wait()
        pltpu.sync_copy(rows, acc.at[didx], add=True)

    @pl.loop(0, _ZROWS)
    def _zr(r):
        @pl.loop(0, _H, step=16)
        def _zc(c):
            zv[r, pl.ds(c, 16)] = jnp.zeros((16,), jnp.float32)

    @pl.loop(0, _RPS, step=_ZROWS)
    def _za(r):
        pltpu.sync_copy(zv, acc.at[pl.ds(row0 + r, _ZROWS)])

    plsc.subcore_barrier()

    idx_start(ebase, sA, dA, semSA, semDA)
    idx_start(ebase + _CHUNK, sB, dB, semSB, semDB)

    @pl.loop(0, _STEPS - 2, step=2)
    def _main(s):
        off = ebase + s * _CHUNK
        process(off, sA, dA, semSA, semDA)
        idx_start(off + 2 * _CHUNK, sA, dA, semSA, semDA)
        process(off + _CHUNK, sB, dB, semSB, semDB)
        idx_start(off + 3 * _CHUNK, sB, dB, semSB, semDB)

    off_t = ebase + (_STEPS - 2) * _CHUNK
    process(off_t, sA, dA, semSA, semDA)
    process(off_t + _CHUNK, sB, dB, semSB, semDB)

    plsc.subcore_barrier()

    pltpu.sync_copy(acc.at[pl.ds(row0, _RPS)],
                    out_hbm.at[cid, pl.ds(row0, _RPS)])


@jax.jit
def _sc_partials(h, src1, dst1):
    kern = pl.kernel(
        _sc_partials_body,
        out_type=jax.ShapeDtypeStruct((_NC, _NP, _H), jnp.float32),
        mesh=plsc.VectorSubcoreMesh(core_axis_name="c", subcore_axis_name="s"),
        scratch_types=[
            pltpu.VMEM((_CHUNK,), jnp.int32),
            pltpu.VMEM((_CHUNK,), jnp.int32),
            pltpu.VMEM((_CHUNK,), jnp.int32),
            pltpu.VMEM((_CHUNK,), jnp.int32),
            pltpu.VMEM((_CHUNK, _H), jnp.float32),
            pltpu.VMEM((_ZROWS, _H), jnp.float32),
            pltpu.VMEM_SHARED((_NP, _H), jnp.float32),
            pltpu.SemaphoreType.DMA,
            pltpu.SemaphoreType.DMA,
            pltpu.SemaphoreType.DMA,
            pltpu.SemaphoreType.DMA,
            pltpu.SemaphoreType.DMA,
        ],
    )
    return kern(h, src1, dst1)


def _enc_body(x_ref, w_ref, b_ref, o_ref):
    o_ref[...] = (
        jnp.dot(x_ref[...], w_ref[...], preferred_element_type=jnp.float32)
        + b_ref[...]
    )


@jax.jit
def _encode(x, Wenc, benc):
    return pl.pallas_call(
        _enc_body,
        out_shape=jax.ShapeDtypeStruct((_N, _H), jnp.float32),
    )(x, Wenc, benc.reshape(1, _H))


def _bn_relu(z, g, b):
    m = jnp.mean(z, axis=0, keepdims=True)
    v = jnp.mean(jnp.square(z - m), axis=0, keepdims=True)
    z = g * (z - m) / jnp.sqrt(v + 1e-5) + b
    return jnp.maximum(z, 0.0)


def _gin_mlp(h_ref, p_ref, w1_ref, b1_ref, gm_ref, bm_ref,
             w2_ref, b2_ref, go_ref, bo_ref, sc_ref):
    h = h_ref[...]
    z = sc_ref[...] * h + (p_ref[0, : _N] + p_ref[1, : _N])
    z = jnp.dot(z, w1_ref[...], preferred_element_type=jnp.float32) + b1_ref[...]
    z = _bn_relu(z, gm_ref[...], bm_ref[...])
    z = jnp.dot(z, w2_ref[...], preferred_element_type=jnp.float32) + b2_ref[...]
    z = _bn_relu(z, go_ref[...], bo_ref[...])
    return z + h


def _layer_body(h_ref, p_ref, w1_ref, b1_ref, gm_ref, bm_ref,
                w2_ref, b2_ref, go_ref, bo_ref, sc_ref, o_ref):
    o_ref[...] = _gin_mlp(h_ref, p_ref, w1_ref, b1_ref, gm_ref, bm_ref,
                          w2_ref, b2_ref, go_ref, bo_ref, sc_ref)


def _last_body(h_ref, p_ref, w1_ref, b1_ref, gm_ref, bm_ref,
               w2_ref, b2_ref, go_ref, bo_ref, sc_ref,
               wo_ref, bo2_ref, o_ref):
    hn = _gin_mlp(h_ref, p_ref, w1_ref, b1_ref, gm_ref, bm_ref,
                  w2_ref, b2_ref, go_ref, bo_ref, sc_ref)
    o_ref[...] = (
        jnp.dot(hn, wo_ref[...], preferred_element_type=jnp.float32)
        + bo2_ref[...]
    )


@jax.jit
def _layer(*args):
    return pl.pallas_call(
        _layer_body,
        out_shape=jax.ShapeDtypeStruct((_N, _H), jnp.float32),
    )(*args)


@jax.jit
def _last(*args):
    return pl.pallas_call(
        _last_body,
        out_shape=jax.ShapeDtypeStruct((_N, 1), jnp.float32),
    )(*args)


def kernel(x, edge_index, Wenc, benc, W1, b1, g_mid, bt_mid, W2, b2,
           eps, g_out, bt_out, Wout, bout):
    pad = _EPWP - _EPW
    src1 = jnp.concatenate(
        [edge_index[0].astype(jnp.int32).reshape(_NW, _EPW),
         jnp.zeros((_NW, pad), jnp.int32)], axis=1).reshape(_NW * _EPWP)
    dst1 = jnp.concatenate(
        [edge_index[1].astype(jnp.int32).reshape(_NW, _EPW),
         jnp.full((_NW, pad), _N, jnp.int32)], axis=1).reshape(_NW * _EPWP)
    h = _encode(x, Wenc, benc)
    for i in range(_L):
        parts = _sc_partials(h, src1, dst1)
        sc = (1.0 + eps[i]) * jnp.ones((1, _H), jnp.float32)
        args = (h, parts, W1[i], b1[i].reshape(1, -1),
                g_mid[i].reshape(1, -1), bt_mid[i].reshape(1, -1),
                W2[i], b2[i].reshape(1, -1),
                g_out[i].reshape(1, -1), bt_out[i].reshape(1, -1), sc)
        if i < _L - 1:
            h = _layer(*args)
        else:
            out = _last(*args, Wout, bout.reshape(1, 1))
    return out


# speedup vs baseline: 1.8847x; 1.1792x over previous
import functools

import jax
import jax.numpy as jnp
from jax import lax
from jax.experimental import pallas as pl
from jax.experimental.pallas import tpu as pltpu
from jax.experimental.pallas import tpu_sc as plsc

_N = 10000
_E = 320000
_D = 128
_H = 128
_L = 3

_NC = 2
_NS = 16
_NW = _NC * _NS
_EPW = _E // _NW
_CHUNK = 80
_STEPS = 126
_EPWP = _STEPS * _CHUNK
_NP = 10240
_RPS = _NP // _NS
_ZROWS = 128


def _sc_partials_body(h_hbm, src_hbm, dst_hbm, out_hbm,
                      s0, d0, s1, d1, s2, d2, s3, d3,
                      rowsA, rowsB, zv, acc,
                      gsA, gsB, ssA, ssB,
                      iS0, iS1, iS2, iS3, iD0, iD1, iD2, iD3):
    cid = lax.axis_index("c")
    sid = lax.axis_index("s")
    wid = sid * _NC + cid
    row0 = sid * _RPS
    ebase = wid * _EPWP

    sidx = [s0, s1, s2, s3]
    didx = [d0, d1, d2, d3]
    isS = [iS0, iS1, iS2, iS3]
    isD = [iD0, iD1, iD2, iD3]
    rows = [rowsA, rowsB]
    gsem = [gsA, gsB]
    ssem = [ssA, ssB]

    def idx_start(c, j):
        off = ebase + c * _CHUNK
        pltpu.async_copy(src_hbm.at[pl.ds(off, _CHUNK)], sidx[j], isS[j])
        pltpu.async_copy(dst_hbm.at[pl.ds(off, _CHUNK)], didx[j], isD[j])

    def idx_wait(c, j):
        off = ebase + c * _CHUNK
        pltpu.make_async_copy(src_hbm.at[pl.ds(off, _CHUNK)], sidx[j],
                              isS[j]).wait()
        pltpu.make_async_copy(dst_hbm.at[pl.ds(off, _CHUNK)], didx[j],
                              isD[j]).wait()

    def gather_start(x, j):
        pltpu.async_copy(h_hbm.at[sidx[j]], rows[x], gsem[x])

    def gather_wait(x, j):
        pltpu.make_async_copy(h_hbm.at[sidx[j]], rows[x], gsem[x]).wait()

    def scatter_start(x, j):
        pltpu.async_copy(rows[x], acc.at[didx[j]], ssem[x], add=True)

    def scatter_wait(x, j):
        pltpu.make_async_copy(rows[x], acc.at[didx[j]], ssem[x]).wait()

    def phase(c, cj, first=False, last=False, noidx=False):
        x, y = cj % 2, (cj + 1) % 2
        j, jp, jn = cj % 4, (cj - 1) % 4, (cj + 1) % 4
        gather_wait(x, j)
        if not first:
            scatter_wait(y, jp)
        if not (last or noidx):
            idx_start(c + 3, jp)
        scatter_start(x, j)
        if not last:
            idx_wait(c + 1, jn)
            gather_start(y, jn)

    @pl.loop(0, _ZROWS)
    def _zr(r):
        @pl.loop(0, _H, step=16)
        def _zc(c):
            zv[r, pl.ds(c, 16)] = jnp.zeros((16,), jnp.float32)

    @pl.loop(0, _RPS, step=_ZROWS)
    def _za(r):
        pltpu.sync_copy(zv, acc.at[pl.ds(row0 + r, _ZROWS)])

    plsc.subcore_barrier()

    idx_start(0, 0)
    idx_start(1, 1)
    idx_start(2, 2)
    idx_wait(0, 0)
    gather_start(0, 0)

    phase(0, 0, first=True)
    phase(1, 1)

    @pl.loop(2, _STEPS - 4, step=4)
    def _main(s):
        phase(s, 2)
        phase(s + 1, 3)
        phase(s + 2, 0)
        phase(s + 3, 1)

    phase(_STEPS - 4, 2)
    phase(_STEPS - 3, 3, noidx=True)
    phase(_STEPS - 2, 0, last=True)
    idx_wait(_STEPS - 1, 1)
    gather_start(1, 1)
    phase(_STEPS - 1, 1, last=True)
    scatter_wait(1, 1)

    plsc.subcore_barrier()

    pltpu.sync_copy(acc.at[pl.ds(row0, _RPS)],
                    out_hbm.at[cid, pl.ds(row0, _RPS)])


@jax.jit
def _sc_partials(h, src1, dst1):
    kern = pl.kernel(
        _sc_partials_body,
        out_type=jax.ShapeDtypeStruct((_NC, _NP, _H), jnp.float32),
        mesh=plsc.VectorSubcoreMesh(core_axis_name="c", subcore_axis_name="s"),
        scratch_types=(
            [pltpu.VMEM((_CHUNK,), jnp.int32)] * 8
            + [pltpu.VMEM((_CHUNK, _H), jnp.float32)] * 2
            + [pltpu.VMEM((_ZROWS, _H), jnp.float32),
               pltpu.VMEM_SHARED((_NP, _H), jnp.float32)]
            + [pltpu.SemaphoreType.DMA] * 12
        ),
    )
    return kern(h, src1, dst1)


def _enc_body(x_ref, w_ref, b_ref, o_ref):
    o_ref[...] = (
        jnp.dot(x_ref[...], w_ref[...], preferred_element_type=jnp.float32)
        + b_ref[...]
    )


@jax.jit
def _encode(x, Wenc, benc):
    return pl.pallas_call(
        _enc_body,
        out_shape=jax.ShapeDtypeStruct((_N, _H), jnp.float32),
    )(x, Wenc, benc.reshape(1, _H))


def _bn_relu(z, g, b):
    m = jnp.mean(z, axis=0, keepdims=True)
    v = jnp.mean(jnp.square(z - m), axis=0, keepdims=True)
    z = g * (z - m) / jnp.sqrt(v + 1e-5) + b
    return jnp.maximum(z, 0.0)


def _gin_mlp(h_ref, p_ref, w1_ref, b1_ref, gm_ref, bm_ref,
             w2_ref, b2_ref, go_ref, bo_ref, sc_ref):
    h = h_ref[...]
    z = sc_ref[...] * h + (p_ref[0, : _N] + p_ref[1, : _N])
    z = jnp.dot(z, w1_ref[...], preferred_element_type=jnp.float32) + b1_ref[...]
    z = _bn_relu(z, gm_ref[...], bm_ref[...])
    z = jnp.dot(z, w2_ref[...], preferred_element_type=jnp.float32) + b2_ref[...]
    z = _bn_relu(z, go_ref[...], bo_ref[...])
    return z + h


def _layer_body(h_ref, p_ref, w1_ref, b1_ref, gm_ref, bm_ref,
                w2_ref, b2_ref, go_ref, bo_ref, sc_ref, o_ref):
    o_ref[...] = _gin_mlp(h_ref, p_ref, w1_ref, b1_ref, gm_ref, bm_ref,
                          w2_ref, b2_ref, go_ref, bo_ref, sc_ref)


def _last_body(h_ref, p_ref, w1_ref, b1_ref, gm_ref, bm_ref,
               w2_ref, b2_ref, go_ref, bo_ref, sc_ref,
               wo_ref, bo2_ref, o_ref):
    hn = _gin_mlp(h_ref, p_ref, w1_ref, b1_ref, gm_ref, bm_ref,
                  w2_ref, b2_ref, go_ref, bo_ref, sc_ref)
    o_ref[...] = (
        jnp.dot(hn, wo_ref[...], preferred_element_type=jnp.float32)
        + bo2_ref[...]
    )


@jax.jit
def _layer(*args):
    return pl.pallas_call(
        _layer_body,
        out_shape=jax.ShapeDtypeStruct((_N, _H), jnp.float32),
    )(*args)


@jax.jit
def _last(*args):
    return pl.pallas_call(
        _last_body,
        out_shape=jax.ShapeDtypeStruct((_N, 1), jnp.float32),
    )(*args)


def kernel(x, edge_index, Wenc, benc, W1, b1, g_mid, bt_mid, W2, b2,
           eps, g_out, bt_out, Wout, bout):
    pad = _EPWP - _EPW
    src1 = jnp.concatenate(
        [edge_index[0].astype(jnp.int32).reshape(_NW, _EPW),
         jnp.zeros((_NW, pad), jnp.int32)], axis=1).reshape(_NW * _EPWP)
    dst1 = jnp.concatenate(
        [edge_index[1].astype(jnp.int32).reshape(_NW, _EPW),
         jnp.full((_NW, pad), _N, jnp.int32)], axis=1).reshape(_NW * _EPWP)
    h = _encode(x, Wenc, benc)
    for i in range(_L):
        parts = _sc_partials(h, src1, dst1)
        sc = (1.0 + eps[i]) * jnp.ones((1, _H), jnp.float32)
        args = (h, parts, W1[i], b1[i].reshape(1, -1),
                g_mid[i].reshape(1, -1), bt_mid[i].reshape(1, -1),
                W2[i], b2[i].reshape(1, -1),
                g_out[i].reshape(1, -1), bt_out[i].reshape(1, -1), sc)
        if i < _L - 1:
            h = _layer(*args)
        else:
            out = _last(*args, Wout, bout.reshape(1, 1))
    return out
